# Initial kernel scaffold; baseline (speedup 1.0000x reference)
import functools

import jax
import jax.numpy as jnp
from jax import lax
from jax.experimental import pallas as pl
from jax.experimental.pallas import tpu as pltpu
from jax.experimental.pallas import tpu_sc as plsc

RES_ = 128
FDIM_ = 16
ORIG = 63
SUB = RES_ - ORIG
CHUNK = 128
NWORK = 32
L = 16

_CORNER = [(dz * SUB + dy) * SUB + dx
           for dz in (0, 1) for dy in (0, 1) for dx in (0, 1)]


def _make_sc_call(num_pts):
    nchunks = num_pts // CHUNK
    mesh = plsc.VectorSubcoreMesh(core_axis_name="c", subcore_axis_name="s")

    @functools.partial(
        pl.kernel,
        out_type=jax.ShapeDtypeStruct((num_pts, FDIM_), jnp.float32),
        mesh=mesh,
        scratch_types=[
            pltpu.VMEM((3, CHUNK), jnp.float32),
            pltpu.VMEM((8, CHUNK), jnp.int32),
            pltpu.VMEM((8, CHUNK), jnp.float32),
            pltpu.VMEM((8, CHUNK, FDIM_), jnp.float32),
            pltpu.VMEM((CHUNK, FDIM_), jnp.float32),
            pltpu.SemaphoreType.DMA,
        ],
    )
    def sc_fn(xs, ys, zs, table, out, pts_v, idx_v, w_v, rows_v, out_v, sem):
        ncores = mesh.num_cores
        wid = lax.axis_index("s") * ncores + lax.axis_index("c")
        my_n = (nchunks - wid + (NWORK - 1)) // NWORK

        def chunk_body(g, _):
            off = (wid + g * NWORK) * CHUNK

            pltpu.sync_copy(xs.at[pl.ds(off, CHUNK)], pts_v.at[0])
            pltpu.sync_copy(ys.at[pl.ds(off, CHUNK)], pts_v.at[1])
            pltpu.sync_copy(zs.at[pl.ds(off, CHUNK)], pts_v.at[2])

            for i in range(CHUNK // L):
                sl = pl.ds(i * L, L)
                px = pts_v[0, sl]
                py = pts_v[1, sl]
                pz = pts_v[2, sl]
                fx = (px + 1.0) * 0.5 * (RES_ - 1)
                fy = (py + 1.0) * 0.5 * (RES_ - 1)
                fz = (pz + 1.0) * 0.5 * (RES_ - 1)
                xi = jnp.minimum(fx.astype(jnp.int32), RES_ - 2)
                yi = jnp.minimum(fy.astype(jnp.int32), RES_ - 2)
                zi = jnp.minimum(fz.astype(jnp.int32), RES_ - 2)
                tx = fx - xi.astype(jnp.float32)
                ty = fy - yi.astype(jnp.float32)
                tz = fz - zi.astype(jnp.float32)
                base = ((zi - ORIG) * SUB + (yi - ORIG)) * SUB + (xi - ORIG)
                ux = 1.0 - tx
                uy = 1.0 - ty
                uz = 1.0 - tz
                wzy = [uz * uy, uz * ty, tz * uy, tz * ty]
                for c in range(8):
                    idx_v[c, sl] = base + _CORNER[c]
                    w_v[c, sl] = wzy[c // 2] * (tx if (c & 1) else ux)

            copies = [
                pltpu.make_async_copy(table.at[idx_v.at[c]], rows_v.at[c], sem)
                for c in range(8)
            ]
            for cp in copies:
                cp.start()
            for cp in copies:
                cp.wait()

            def pt_body(j, _):
                acc = w_v[0, j] * rows_v[0, j, :]
                for c in range(1, 8):
                    acc = acc + w_v[c, j] * rows_v[c, j, :]
                out_v[j, :] = acc
                return 0

            lax.fori_loop(0, CHUNK, pt_body, 0, unroll=2)

            pltpu.sync_copy(out_v, out.at[pl.ds(off, CHUNK), :])
            return 0

        lax.fori_loop(0, my_n, chunk_body, 0)

    return sc_fn


def kernel(points, modality_idx, grid):
    del modality_idx
    B, N, _ = points.shape
    num_pts = B * N
    assert num_pts % CHUNK == 0
    assert grid.shape == (FDIM_, RES_, RES_, RES_)

    pts = points.reshape(num_pts, 3)
    xs = jnp.ascontiguousarray(pts[:, 0])
    ys = jnp.ascontiguousarray(pts[:, 1])
    zs = jnp.ascontiguousarray(pts[:, 2])
    sub = lax.slice(grid, (0, ORIG, ORIG, ORIG), (FDIM_, RES_, RES_, RES_))
    table = jnp.transpose(sub, (1, 2, 3, 0)).reshape(SUB * SUB * SUB, FDIM_)

    feats = _make_sc_call(num_pts)(xs, ys, zs, table)
    return feats.reshape(B, N, FDIM_)


# speedup vs baseline: 2.7870x; 2.7870x over previous
import functools

import jax
import jax.numpy as jnp
from jax import lax
from jax.experimental import pallas as pl
from jax.experimental.pallas import tpu as pltpu
from jax.experimental.pallas import tpu_sc as plsc

RES_ = 128
FDIM_ = 16
ORIG = 63
SUB = RES_ - ORIG
CHUNK = 128
NWORK = 32
L = 16

_CORNER = [(dz * SUB + dy) * SUB + dx
           for dz in (0, 1) for dy in (0, 1) for dx in (0, 1)]


def _make_sc_call(num_pts):
    nchunks = num_pts // CHUNK
    mesh = plsc.VectorSubcoreMesh(core_axis_name="c", subcore_axis_name="s")

    @functools.partial(
        pl.kernel,
        out_type=jax.ShapeDtypeStruct((num_pts, FDIM_), jnp.float32),
        mesh=mesh,
        scratch_types=[
            pltpu.VMEM((3, CHUNK), jnp.float32),
            pltpu.VMEM((8, CHUNK), jnp.int32),
            pltpu.VMEM((8, CHUNK), jnp.float32),
            pltpu.VMEM((8, CHUNK, FDIM_), jnp.float32),
            pltpu.VMEM((CHUNK, FDIM_), jnp.float32),
            pltpu.SemaphoreType.DMA,
        ],
        compiler_params=pltpu.CompilerParams(use_tc_tiling_on_sc=False),
    )
    def sc_fn(xs, ys, zs, table, out, pts_v, idx_v, w_v, rows_v, out_v, sem):
        ncores = mesh.num_cores
        wid = lax.axis_index("s") * ncores + lax.axis_index("c")
        my_n = (nchunks - wid + (NWORK - 1)) // NWORK

        def chunk_body(g, _):
            off = (wid + g * NWORK) * CHUNK

            pltpu.sync_copy(xs.at[pl.ds(off, CHUNK)], pts_v.at[0])
            pltpu.sync_copy(ys.at[pl.ds(off, CHUNK)], pts_v.at[1])
            pltpu.sync_copy(zs.at[pl.ds(off, CHUNK)], pts_v.at[2])

            for i in range(CHUNK // L):
                sl = pl.ds(i * L, L)
                px = pts_v[0, sl]
                py = pts_v[1, sl]
                pz = pts_v[2, sl]
                fx = (px + 1.0) * 0.5 * (RES_ - 1)
                fy = (py + 1.0) * 0.5 * (RES_ - 1)
                fz = (pz + 1.0) * 0.5 * (RES_ - 1)
                xi = jnp.minimum(fx.astype(jnp.int32), RES_ - 2)
                yi = jnp.minimum(fy.astype(jnp.int32), RES_ - 2)
                zi = jnp.minimum(fz.astype(jnp.int32), RES_ - 2)
                tx = fx - xi.astype(jnp.float32)
                ty = fy - yi.astype(jnp.float32)
                tz = fz - zi.astype(jnp.float32)
                base = ((zi - ORIG) * SUB + (yi - ORIG)) * SUB + (xi - ORIG)
                ux = 1.0 - tx
                uy = 1.0 - ty
                uz = 1.0 - tz
                wzy = [uz * uy, uz * ty, tz * uy, tz * ty]
                for c in range(8):
                    idx_v[c, sl] = base + _CORNER[c]
                    w_v[c, sl] = wzy[c // 2] * (tx if (c & 1) else ux)

            copies = [
                pltpu.make_async_copy(table.at[idx_v.at[c]], rows_v.at[c], sem)
                for c in range(8)
            ]
            for cp in copies:
                cp.start()
            for cp in copies:
                cp.wait()

            for i in range(CHUNK // L):
                wvecs = [w_v[c, pl.ds(i * L, L)] for c in range(8)]

                def pt_body(jj, _, i=i, wvecs=wvecs):
                    j = i * L + jj
                    sel = jnp.full((L,), jj, jnp.int32)
                    acc = jnp.take_along_axis(wvecs[0], sel, axis=0) \
                        * rows_v[0, j, :]
                    for c in range(1, 8):
                        wb = jnp.take_along_axis(wvecs[c], sel, axis=0)
                        acc = acc + wb * rows_v[c, j, :]
                    out_v[j, :] = acc
                    return 0

                lax.fori_loop(0, L, pt_body, 0, unroll=2)

            pltpu.sync_copy(out_v, out.at[pl.ds(off, CHUNK), :])
            return 0

        lax.fori_loop(0, my_n, chunk_body, 0)

    return sc_fn


def kernel(points, modality_idx, grid):
    del modality_idx
    B, N, _ = points.shape
    num_pts = B * N
    assert num_pts % CHUNK == 0
    assert grid.shape == (FDIM_, RES_, RES_, RES_)

    ptsT = jnp.transpose(points.reshape(num_pts, 3))
    xs, ys, zs = ptsT[0], ptsT[1], ptsT[2]
    sub = lax.slice(grid, (0, ORIG, ORIG, ORIG), (FDIM_, RES_, RES_, RES_))
    table = jnp.transpose(sub, (1, 2, 3, 0)).reshape(SUB * SUB * SUB, FDIM_)

    feats = _make_sc_call(num_pts)(xs, ys, zs, table)
    return feats.reshape(B, N, FDIM_)
